# baseline (speedup 1.0000x reference)
import functools

import jax
import jax.numpy as jnp
from jax import lax
from jax.experimental import pallas as pl
from jax.experimental.pallas import tpu as pltpu
from jax.experimental.pallas import tpu_sc as plsc

B, T, C = 1, 2048, 1024
E, K = 8, 2
FF = 4 * C
CAP = (B * T) // E
FFT = 1024
NFF = FF // FFT
GPAD = 128

_info = plsc.get_sparse_core_info()
_NC, _NS = _info.num_cores, _info.num_subcores
NW = _NC * _NS
BPW = T // NW
CHUNK = 32

_sc_mesh = plsc.VectorSubcoreMesh(core_axis_name="c", subcore_axis_name="s")


def _gate_body(x_ref, wg_ref, bg_ref, o_ref):
    o_ref[...] = jax.nn.sigmoid(
        jnp.dot(x_ref[...], wg_ref[...], preferred_element_type=jnp.float32)
        + bg_ref[...]
    )


@functools.partial(
    pl.kernel,
    mesh=_sc_mesh,
    out_type=jax.ShapeDtypeStruct((T, C), jnp.float32),
    scratch_types=[
        pltpu.VMEM((BPW,), jnp.int32),
        pltpu.VMEM((BPW, C), jnp.float32),
        pltpu.SemaphoreType.DMA,
    ],
)
def _sc_dispatch(table_hbm, idx_hbm, out_hbm, idx_v, rows_v, sem):
    wid = lax.axis_index("s") * _NC + lax.axis_index("c")
    base = wid * BPW
    pltpu.sync_copy(idx_hbm.at[pl.ds(base, BPW)], idx_v)
    pltpu.async_copy(table_hbm.at[idx_v], rows_v, sem).wait()
    pltpu.sync_copy(rows_v, out_hbm.at[pl.ds(base, BPW)])


def _add_body(a_ref, b_ref, o_ref):
    o_ref[...] = a_ref[...] + b_ref[...]


def _ffn_body(sco_ref, xs_ref, w1_ref, b1_ref, w2_ref, b2_ref, o_ref, acc_ref):
    e = pl.program_id(0)
    ff = pl.program_id(1)
    live = e > 0

    @pl.when(jnp.logical_and(live, ff == 0))
    def _():
        acc_ref[...] = jnp.zeros_like(acc_ref)

    @pl.when(live)
    def _():
        h = jnp.dot(xs_ref[...].astype(jnp.bfloat16), w1_ref[0],
                    preferred_element_type=jnp.float32)
        h = jax.nn.gelu(h + b1_ref[0])
        acc_ref[...] += jnp.dot(h.astype(jnp.bfloat16), w2_ref[0],
                                preferred_element_type=jnp.float32)

    @pl.when(ff == NFF - 1)
    def _():
        @pl.when(live)
        def _():
            o_ref[...] = (acc_ref[...] + b2_ref[0]) * sco_ref[...]

        @pl.when(jnp.logical_not(live))
        def _():
            o_ref[...] = jnp.zeros_like(o_ref)


def _routing(scores):
    g_i, idx = jax.lax.top_k(scores, K)
    g_scores = g_i / jnp.sum(g_i, axis=-1, keepdims=True)
    sti = jnp.argsort(-g_scores[:, 0])
    sind = jnp.take_along_axis(idx, sti[:, None], axis=0)
    ssc = jnp.take_along_axis(g_scores, sti[:, None], axis=0)
    flat_ind = jnp.swapaxes(sind, 0, 1).reshape(-1)
    flat_sc = jnp.swapaxes(ssc, 0, 1).reshape(-1)
    oh = jax.nn.one_hot(flat_ind, E, dtype=jnp.int32)
    pie = jnp.cumsum(oh, axis=0) * oh
    tokens_per_expert = jnp.max(pie, axis=0) / (B * T)
    esc = flat_sc[:, None] * oh
    pie_k = pie.reshape(K, T, E)
    esc_k = esc.reshape(K, T, E)
    pos_s = jnp.max(jnp.swapaxes(pie_k, 0, 1), axis=1) - 1
    sc_s = jnp.max(jnp.swapaxes(esc_k, 0, 1), axis=1)
    kept = (pos_s >= 0) & (pos_s < CAP)
    col = jnp.where(kept, pos_s, CAP)
    ee = jnp.broadcast_to(jnp.arange(E)[None, :], (T, E))
    tt = jnp.broadcast_to(sti[:, None], (T, E))
    tok = jnp.zeros((E, CAP + 1), jnp.int32).at[ee, col].set(tt)[:, :CAP]
    sco = jnp.zeros((E, CAP + 1), jnp.float32).at[ee, col].set(
        jnp.where(kept, sc_s, 0.0))[:, :CAP]
    a_list = []
    for k in range(K):
        pos_k = jnp.max(pie_k[k], axis=-1) - 1
        slot_k = jnp.where(pos_k < CAP, (sind[:, k] + 1) * CAP + pos_k, 0)
        a_list.append(jnp.zeros((T,), jnp.int32).at[sti].set(slot_k))
    sn = scores / jnp.sum(scores, axis=-1, keepdims=True)
    sn = jnp.take_along_axis(sn, idx, axis=-1)
    ohf = jax.nn.one_hot(idx.reshape(-1), E, dtype=jnp.float32)
    f = jnp.sum(ohf, axis=0) / (B * T)
    p = jnp.sum(ohf * sn.reshape(-1)[:, None], axis=0) / (B * T)
    return tok, sco, a_list[0], a_list[1], tokens_per_expert, f, p


def kernel(x, W_shared, b_shared, W_gate, b_gate, W1, b1, W2, b2):
    xf = x.reshape(T, C)
    Wg = jnp.zeros((C, GPAD), x.dtype).at[:, :E].set(W_gate)
    bg = jnp.zeros((1, GPAD), x.dtype).at[0, :E].set(b_gate)
    scores_pad = pl.pallas_call(
        _gate_body,
        out_shape=jax.ShapeDtypeStruct((T, GPAD), jnp.float32),
    )(xf, Wg, bg)
    scores = scores_pad[:, :E]

    tok, sco, a0, a1, tokens_per_expert, f, p = _routing(scores)

    x_sorted = _sc_dispatch(xf, tok.reshape(-1))

    def _em1(e, f_):
        return (jnp.maximum(e - 1, 0), 0)

    hs = pl.pallas_call(
        _ffn_body,
        grid=(E + 1, NFF),
        in_specs=[
            pl.BlockSpec((CAP, 1), _em1),
            pl.BlockSpec((CAP, C), _em1),
            pl.BlockSpec((1, C, FFT),
                         lambda e, f_: (jnp.maximum(e - 1, 0), 0, f_)),
            pl.BlockSpec((1, 1, FFT),
                         lambda e, f_: (jnp.maximum(e - 1, 0), 0, f_)),
            pl.BlockSpec((1, FFT, C),
                         lambda e, f_: (jnp.maximum(e - 1, 0), f_, 0)),
            pl.BlockSpec((1, 1, C),
                         lambda e, f_: (jnp.maximum(e - 1, 0), 0, 0)),
        ],
        out_specs=pl.BlockSpec((CAP, C), lambda e, f_: (e, 0)),
        out_shape=jax.ShapeDtypeStruct(((E + 1) * CAP, C), jnp.float32),
        scratch_shapes=[pltpu.VMEM((CAP, C), jnp.float32)],
        compiler_params=pltpu.CompilerParams(
            dimension_semantics=("arbitrary", "arbitrary")),
    )(sco.reshape(E * CAP, 1), x_sorted, W1.astype(jnp.bfloat16),
      b1.reshape(E, 1, FF), W2.astype(jnp.bfloat16), b2.reshape(E, 1, C))

    g0 = _sc_dispatch(hs, a0)
    g1 = _sc_dispatch(hs, a1)
    RB = T // 4
    out = pl.pallas_call(
        _add_body,
        grid=(4,),
        in_specs=[
            pl.BlockSpec((RB, C), lambda i: (i, 0)),
            pl.BlockSpec((RB, C), lambda i: (i, 0)),
        ],
        out_specs=pl.BlockSpec((RB, C), lambda i: (i, 0)),
        out_shape=jax.ShapeDtypeStruct((T, C), jnp.float32),
    )(g0, g1)

    return out.reshape(B, T, C), tokens_per_expert, f, p


# speedup vs baseline: 1.6391x; 1.6391x over previous
import functools

import jax
import jax.numpy as jnp
from jax import lax
from jax.experimental import pallas as pl
from jax.experimental.pallas import tpu as pltpu
from jax.experimental.pallas import tpu_sc as plsc

B, T, C = 1, 2048, 1024
E, K = 8, 2
FF = 4 * C
CAP = (B * T) // E
FFT = 1024
NFF = FF // FFT
GPAD = 128

_info = plsc.get_sparse_core_info()
_NC, _NS = _info.num_cores, _info.num_subcores
NW = _NC * _NS
BPW = T // NW

_sc_mesh = plsc.VectorSubcoreMesh(core_axis_name="c", subcore_axis_name="s")


def _gate_body(x_ref, wg_ref, bg_ref, o_ref):
    o_ref[...] = jax.nn.sigmoid(
        jnp.dot(x_ref[...], wg_ref[...], preferred_element_type=jnp.float32)
        + bg_ref[...]
    )


@functools.partial(
    pl.kernel,
    mesh=_sc_mesh,
    out_type=jax.ShapeDtypeStruct((T, C), jnp.float32),
    scratch_types=[
        pltpu.VMEM((BPW,), jnp.int32),
        pltpu.VMEM((BPW, C), jnp.float32),
        pltpu.SemaphoreType.DMA,
    ],
)
def _sc_dispatch(table_hbm, idx_hbm, out_hbm, idx_v, rows_v, sem):
    wid = lax.axis_index("s") * _NC + lax.axis_index("c")
    base = wid * BPW
    pltpu.sync_copy(idx_hbm.at[pl.ds(base, BPW)], idx_v)
    pltpu.async_copy(table_hbm.at[idx_v], rows_v, sem).wait()
    pltpu.sync_copy(rows_v, out_hbm.at[pl.ds(base, BPW)])


def _ffn_body(tok_ref, sco_ref, xs_ref, w1_ref, b1_ref, w2_ref, b2_ref,
              o_ref, acc_ref):
    e = pl.program_id(0)
    ff = pl.program_id(1)

    @pl.when(jnp.logical_and(e == 0, ff == 0))
    def _():
        o_ref[...] = jnp.zeros_like(o_ref)

    @pl.when(ff == 0)
    def _():
        acc_ref[...] = jnp.zeros_like(acc_ref)

    h = jnp.dot(xs_ref[...].astype(jnp.bfloat16),
                w1_ref[0].astype(jnp.bfloat16),
                preferred_element_type=jnp.float32)
    h = jax.nn.gelu(h + b1_ref[0])
    acc_ref[...] += jnp.dot(h.astype(jnp.bfloat16),
                            w2_ref[0].astype(jnp.bfloat16),
                            preferred_element_type=jnp.float32)

    @pl.when(ff == NFF - 1)
    def _():
        acc_ref[...] += b2_ref[0]

        def scatter(i, carry):
            t = tok_ref[e, i]
            s = sco_ref[e, i]
            o_ref[pl.ds(t, 1), :] += acc_ref[pl.ds(i, 1), :] * s
            return carry

        lax.fori_loop(0, CAP, scatter, 0)


def _routing(scores):
    g_i, idx = jax.lax.top_k(scores, K)
    g_scores = g_i / jnp.sum(g_i, axis=-1, keepdims=True)
    sti = jnp.argsort(-g_scores[:, 0])
    sind = jnp.take_along_axis(idx, sti[:, None], axis=0)
    ssc = jnp.take_along_axis(g_scores, sti[:, None], axis=0)
    flat_ind = jnp.swapaxes(sind, 0, 1).reshape(-1)
    flat_sc = jnp.swapaxes(ssc, 0, 1).reshape(-1)
    oh = jax.nn.one_hot(flat_ind, E, dtype=jnp.int32)
    pie = jnp.cumsum(oh, axis=0) * oh
    tokens_per_expert = jnp.max(pie, axis=0) / (B * T)
    esc = flat_sc[:, None] * oh
    pos_s = jnp.max(jnp.swapaxes(pie.reshape(K, T, E), 0, 1), axis=1) - 1
    sc_s = jnp.max(jnp.swapaxes(esc.reshape(K, T, E), 0, 1), axis=1)
    kept = (pos_s >= 0) & (pos_s < CAP)
    col = jnp.where(kept, pos_s, CAP)
    ee = jnp.broadcast_to(jnp.arange(E)[None, :], (T, E))
    tt = jnp.broadcast_to(sti[:, None], (T, E))
    tok = jnp.zeros((E, CAP + 1), jnp.int32).at[ee, col].set(tt)[:, :CAP]
    sco = jnp.zeros((E, CAP + 1), jnp.float32).at[ee, col].set(
        jnp.where(kept, sc_s, 0.0))[:, :CAP]
    sn = scores / jnp.sum(scores, axis=-1, keepdims=True)
    sn = jnp.take_along_axis(sn, idx, axis=-1)
    ohf = jax.nn.one_hot(idx.reshape(-1), E, dtype=jnp.float32)
    f = jnp.sum(ohf, axis=0) / (B * T)
    p = jnp.sum(ohf * sn.reshape(-1)[:, None], axis=0) / (B * T)
    return tok, sco, tokens_per_expert, f, p


def kernel(x, W_shared, b_shared, W_gate, b_gate, W1, b1, W2, b2):
    xf = x.reshape(T, C)
    Wg = jnp.zeros((C, GPAD), x.dtype).at[:, :E].set(W_gate)
    bg = jnp.zeros((1, GPAD), x.dtype).at[0, :E].set(b_gate)
    scores_pad = pl.pallas_call(
        _gate_body,
        out_shape=jax.ShapeDtypeStruct((T, GPAD), jnp.float32),
    )(xf, Wg, bg)
    scores = scores_pad[:, :E]

    tok, sco, tokens_per_expert, f, p = _routing(scores)

    x_sorted = _sc_dispatch(xf, tok.reshape(-1))

    out = pl.pallas_call(
        _ffn_body,
        grid=(E, NFF),
        in_specs=[
            pl.BlockSpec(memory_space=pltpu.SMEM),
            pl.BlockSpec(memory_space=pltpu.SMEM),
            pl.BlockSpec((CAP, C), lambda e, f_: (e, 0)),
            pl.BlockSpec((1, C, FFT), lambda e, f_: (e, 0, f_)),
            pl.BlockSpec((1, 1, FFT), lambda e, f_: (e, 0, f_)),
            pl.BlockSpec((1, FFT, C), lambda e, f_: (e, f_, 0)),
            pl.BlockSpec((1, 1, C), lambda e, f_: (e, 0, 0)),
        ],
        out_specs=pl.BlockSpec((T, C), lambda e, f_: (0, 0)),
        out_shape=jax.ShapeDtypeStruct((T, C), jnp.float32),
        scratch_shapes=[pltpu.VMEM((CAP, C), jnp.float32)],
        compiler_params=pltpu.CompilerParams(
            dimension_semantics=("arbitrary", "arbitrary")),
    )(tok, sco, x_sorted, W1, b1.reshape(E, 1, FF), W2, b2.reshape(E, 1, C))

    return out.reshape(B, T, C), tokens_per_expert, f, p


# speedup vs baseline: 1.7541x; 1.0702x over previous
import functools

import jax
import jax.numpy as jnp
from jax import lax
from jax.experimental import pallas as pl
from jax.experimental.pallas import tpu as pltpu
from jax.experimental.pallas import tpu_sc as plsc

B, T, C = 1, 2048, 1024
E, K = 8, 2
FF = 4 * C
CAP = (B * T) // E
FFT = 1024
NFF = FF // FFT
GPAD = 128

_info = plsc.get_sparse_core_info()
_NC, _NS = _info.num_cores, _info.num_subcores
NW = _NC * _NS
BPW = T // NW

_sc_mesh = plsc.VectorSubcoreMesh(core_axis_name="c", subcore_axis_name="s")


def _gate_body(x_ref, wg_ref, bg_ref, o_ref):
    o_ref[...] = jax.nn.sigmoid(
        jnp.dot(x_ref[...], wg_ref[...], preferred_element_type=jnp.float32)
        + bg_ref[...]
    )


@functools.partial(
    pl.kernel,
    mesh=_sc_mesh,
    out_type=jax.ShapeDtypeStruct((T, C), jnp.float32),
    scratch_types=[
        pltpu.VMEM((BPW,), jnp.int32),
        pltpu.VMEM((BPW, C), jnp.float32),
        pltpu.SemaphoreType.DMA,
    ],
)
def _sc_dispatch(table_hbm, idx_hbm, out_hbm, idx_v, rows_v, sem):
    wid = lax.axis_index("s") * _NC + lax.axis_index("c")
    base = wid * BPW
    pltpu.sync_copy(idx_hbm.at[pl.ds(base, BPW)], idx_v)
    pltpu.async_copy(table_hbm.at[idx_v], rows_v, sem).wait()
    pltpu.sync_copy(rows_v, out_hbm.at[pl.ds(base, BPW)])


def _ffn_body(tokv_ref, scov_ref, xs_ref, w1_ref, b1_ref, w2_ref, b2_ref,
              o_ref, acc_ref):
    e = pl.program_id(0)
    ff = pl.program_id(1)

    @pl.when(jnp.logical_and(e == 0, ff == 0))
    def _():
        o_ref[...] = jnp.zeros_like(o_ref)

    @pl.when(ff == 0)
    def _():
        acc_ref[...] = jnp.zeros_like(acc_ref)

    h = jnp.dot(xs_ref[...].astype(jnp.bfloat16),
                w1_ref[0].astype(jnp.bfloat16),
                preferred_element_type=jnp.float32)
    h = jax.nn.gelu(h + b1_ref[0])
    acc_ref[...] += jnp.dot(h.astype(jnp.bfloat16),
                            w2_ref[0].astype(jnp.bfloat16),
                            preferred_element_type=jnp.float32)

    @pl.when(ff == NFF - 1)
    def _():
        res = acc_ref[...] + b2_ref[0]
        rows = lax.broadcasted_iota(jnp.int32, (T, CAP), 0)
        P = jnp.where(rows == tokv_ref[0], scov_ref[0], 0.0)
        o_ref[...] += jnp.dot(P.astype(jnp.bfloat16), res.astype(jnp.bfloat16),
                              preferred_element_type=jnp.float32)


def _routing(scores):
    g_i, idx = jax.lax.top_k(scores, K)
    g_scores = g_i / jnp.sum(g_i, axis=-1, keepdims=True)
    sti = jnp.argsort(-g_scores[:, 0])
    sind = jnp.take_along_axis(idx, sti[:, None], axis=0)
    ssc = jnp.take_along_axis(g_scores, sti[:, None], axis=0)
    flat_ind = jnp.swapaxes(sind, 0, 1).reshape(-1)
    flat_sc = jnp.swapaxes(ssc, 0, 1).reshape(-1)
    oh = jax.nn.one_hot(flat_ind, E, dtype=jnp.int32)
    pie = jnp.cumsum(oh, axis=0) * oh
    tokens_per_expert = jnp.max(pie, axis=0) / (B * T)
    esc = flat_sc[:, None] * oh
    pos_s = jnp.max(jnp.swapaxes(pie.reshape(K, T, E), 0, 1), axis=1) - 1
    sc_s = jnp.max(jnp.swapaxes(esc.reshape(K, T, E), 0, 1), axis=1)
    kept = (pos_s >= 0) & (pos_s < CAP)
    col = jnp.where(kept, pos_s, CAP)
    ee = jnp.broadcast_to(jnp.arange(E)[None, :], (T, E))
    tt = jnp.broadcast_to(sti[:, None], (T, E))
    tok = jnp.zeros((E, CAP + 1), jnp.int32).at[ee, col].set(tt)[:, :CAP]
    sco = jnp.zeros((E, CAP + 1), jnp.float32).at[ee, col].set(
        jnp.where(kept, sc_s, 0.0))[:, :CAP]
    sn = scores / jnp.sum(scores, axis=-1, keepdims=True)
    sn = jnp.take_along_axis(sn, idx, axis=-1)
    ohf = jax.nn.one_hot(idx.reshape(-1), E, dtype=jnp.float32)
    f = jnp.sum(ohf, axis=0) / (B * T)
    p = jnp.sum(ohf * sn.reshape(-1)[:, None], axis=0) / (B * T)
    return tok, sco, tokens_per_expert, f, p


def kernel(x, W_shared, b_shared, W_gate, b_gate, W1, b1, W2, b2):
    xf = x.reshape(T, C)
    Wg = jnp.zeros((C, GPAD), x.dtype).at[:, :E].set(W_gate)
    bg = jnp.zeros((1, GPAD), x.dtype).at[0, :E].set(b_gate)
    scores_pad = pl.pallas_call(
        _gate_body,
        out_shape=jax.ShapeDtypeStruct((T, GPAD), jnp.float32),
    )(xf, Wg, bg)
    scores = scores_pad[:, :E]

    tok, sco, tokens_per_expert, f, p = _routing(scores)

    x_sorted = _sc_dispatch(xf, tok.reshape(-1))

    out = pl.pallas_call(
        _ffn_body,
        grid=(E, NFF),
        in_specs=[
            pl.BlockSpec((1, 1, CAP), lambda e, f_: (e, 0, 0)),
            pl.BlockSpec((1, 1, CAP), lambda e, f_: (e, 0, 0)),
            pl.BlockSpec((CAP, C), lambda e, f_: (e, 0)),
            pl.BlockSpec((1, C, FFT), lambda e, f_: (e, 0, f_)),
            pl.BlockSpec((1, 1, FFT), lambda e, f_: (e, 0, f_)),
            pl.BlockSpec((1, FFT, C), lambda e, f_: (e, f_, 0)),
            pl.BlockSpec((1, 1, C), lambda e, f_: (e, 0, 0)),
        ],
        out_specs=pl.BlockSpec((T, C), lambda e, f_: (0, 0)),
        out_shape=jax.ShapeDtypeStruct((T, C), jnp.float32),
        scratch_shapes=[pltpu.VMEM((CAP, C), jnp.float32)],
        compiler_params=pltpu.CompilerParams(
            dimension_semantics=("arbitrary", "arbitrary")),
    )(tok.reshape(E, 1, CAP), sco.reshape(E, 1, CAP), x_sorted, W1,
      b1.reshape(E, 1, FF), W2, b2.reshape(E, 1, C))

    return out.reshape(B, T, C), tokens_per_expert, f, p


# speedup vs baseline: 1.8134x; 1.0338x over previous
import functools

import jax
import jax.numpy as jnp
from jax import lax
from jax.experimental import pallas as pl
from jax.experimental.pallas import tpu as pltpu
from jax.experimental.pallas import tpu_sc as plsc

B, T, C = 1, 2048, 1024
E, K = 8, 2
FF = 4 * C
CAP = (B * T) // E
FFT = 2048
NFF = FF // FFT
GPAD = 128

_info = plsc.get_sparse_core_info()
_NC, _NS = _info.num_cores, _info.num_subcores
NW = _NC * _NS
BPW = T // NW

_sc_mesh = plsc.VectorSubcoreMesh(core_axis_name="c", subcore_axis_name="s")


def _gate_body(x_ref, wg_ref, bg_ref, o_ref):
    o_ref[...] = jax.nn.sigmoid(
        jnp.dot(x_ref[...], wg_ref[...], preferred_element_type=jnp.float32)
        + bg_ref[...]
    )


@functools.partial(
    pl.kernel,
    mesh=_sc_mesh,
    out_type=jax.ShapeDtypeStruct((T, C), jnp.float32),
    scratch_types=[
        pltpu.VMEM((BPW,), jnp.int32),
        pltpu.VMEM((BPW, C), jnp.float32),
        pltpu.SemaphoreType.DMA,
    ],
)
def _sc_dispatch(table_hbm, idx_hbm, out_hbm, idx_v, rows_v, sem):
    wid = lax.axis_index("s") * _NC + lax.axis_index("c")
    base = wid * BPW
    pltpu.sync_copy(idx_hbm.at[pl.ds(base, BPW)], idx_v)
    pltpu.async_copy(table_hbm.at[idx_v], rows_v, sem).wait()
    pltpu.sync_copy(rows_v, out_hbm.at[pl.ds(base, BPW)])


def _ffn_body(tokv_ref, scov_ref, xs_ref, w1_ref, b1_ref, w2_ref, b2_ref,
              o_ref, acc_ref):
    e = pl.program_id(0)
    ff = pl.program_id(1)

    @pl.when(jnp.logical_and(e == 0, ff == 0))
    def _():
        o_ref[...] = jnp.zeros_like(o_ref)

    @pl.when(ff == 0)
    def _():
        acc_ref[...] = jnp.zeros_like(acc_ref)

    h = jnp.dot(xs_ref[...].astype(jnp.bfloat16),
                w1_ref[0].astype(jnp.bfloat16),
                preferred_element_type=jnp.float32)
    h = jax.nn.gelu(h + b1_ref[0])
    acc_ref[...] += jnp.dot(h.astype(jnp.bfloat16),
                            w2_ref[0].astype(jnp.bfloat16),
                            preferred_element_type=jnp.float32)

    @pl.when(ff == NFF - 1)
    def _():
        res = acc_ref[...] + b2_ref[0]
        rows = lax.broadcasted_iota(jnp.int32, (T, CAP), 0)
        P = jnp.where(rows == tokv_ref[0], scov_ref[0], 0.0)
        o_ref[...] += jnp.dot(P.astype(jnp.bfloat16), res.astype(jnp.bfloat16),
                              preferred_element_type=jnp.float32)


def _routing(scores):
    g_i, idx = jax.lax.top_k(scores, K)
    g_scores = g_i / jnp.sum(g_i, axis=-1, keepdims=True)
    sti = jnp.argsort(-g_scores[:, 0])
    sind = jnp.take_along_axis(idx, sti[:, None], axis=0)
    ssc = jnp.take_along_axis(g_scores, sti[:, None], axis=0)
    flat_ind = jnp.swapaxes(sind, 0, 1).reshape(-1)
    flat_sc = jnp.swapaxes(ssc, 0, 1).reshape(-1)
    oh = jax.nn.one_hot(flat_ind, E, dtype=jnp.int32)
    pie = jnp.cumsum(oh, axis=0) * oh
    tokens_per_expert = jnp.max(pie, axis=0) / (B * T)
    esc = flat_sc[:, None] * oh
    pos_s = jnp.max(jnp.swapaxes(pie.reshape(K, T, E), 0, 1), axis=1) - 1
    sc_s = jnp.max(jnp.swapaxes(esc.reshape(K, T, E), 0, 1), axis=1)
    kept = (pos_s >= 0) & (pos_s < CAP)
    col = jnp.where(kept, pos_s, CAP)
    ee = jnp.broadcast_to(jnp.arange(E)[None, :], (T, E))
    tt = jnp.broadcast_to(sti[:, None], (T, E))
    tok = jnp.zeros((E, CAP + 1), jnp.int32).at[ee, col].set(tt)[:, :CAP]
    sco = jnp.zeros((E, CAP + 1), jnp.float32).at[ee, col].set(
        jnp.where(kept, sc_s, 0.0))[:, :CAP]
    sn = scores / jnp.sum(scores, axis=-1, keepdims=True)
    sn = jnp.take_along_axis(sn, idx, axis=-1)
    ohf = jax.nn.one_hot(idx.reshape(-1), E, dtype=jnp.float32)
    f = jnp.sum(ohf, axis=0) / (B * T)
    p = jnp.sum(ohf * sn.reshape(-1)[:, None], axis=0) / (B * T)
    return tok, sco, tokens_per_expert, f, p


def kernel(x, W_shared, b_shared, W_gate, b_gate, W1, b1, W2, b2):
    xf = x.reshape(T, C)
    Wg = jnp.zeros((C, GPAD), x.dtype).at[:, :E].set(W_gate)
    bg = jnp.zeros((1, GPAD), x.dtype).at[0, :E].set(b_gate)
    scores_pad = pl.pallas_call(
        _gate_body,
        out_shape=jax.ShapeDtypeStruct((T, GPAD), jnp.float32),
    )(xf, Wg, bg)
    scores = scores_pad[:, :E]

    tok, sco, tokens_per_expert, f, p = _routing(scores)

    x_sorted = _sc_dispatch(xf, tok.reshape(-1))

    out = pl.pallas_call(
        _ffn_body,
        grid=(E, NFF),
        in_specs=[
            pl.BlockSpec((1, 1, CAP), lambda e, f_: (e, 0, 0)),
            pl.BlockSpec((1, 1, CAP), lambda e, f_: (e, 0, 0)),
            pl.BlockSpec((CAP, C), lambda e, f_: (e, 0)),
            pl.BlockSpec((1, C, FFT), lambda e, f_: (e, 0, f_)),
            pl.BlockSpec((1, 1, FFT), lambda e, f_: (e, 0, f_)),
            pl.BlockSpec((1, FFT, C), lambda e, f_: (e, f_, 0)),
            pl.BlockSpec((1, 1, C), lambda e, f_: (e, 0, 0)),
        ],
        out_specs=pl.BlockSpec((T, C), lambda e, f_: (0, 0)),
        out_shape=jax.ShapeDtypeStruct((T, C), jnp.float32),
        scratch_shapes=[pltpu.VMEM((CAP, C), jnp.float32)],
        compiler_params=pltpu.CompilerParams(
            dimension_semantics=("arbitrary", "arbitrary")),
    )(tok.reshape(E, 1, CAP), sco.reshape(E, 1, CAP), x_sorted, W1,
      b1.reshape(E, 1, FF), W2, b2.reshape(E, 1, C))

    return out.reshape(B, T, C), tokens_per_expert, f, p
